# baseline (speedup 1.0000x reference)
import functools

import numpy as np
import jax
import jax.numpy as jnp
from jax import lax
from jax.experimental import pallas as pl
from jax.experimental.pallas import tpu as pltpu
from jax.experimental.pallas import tpu_sc as plsc

_N = 10000
_NW = 32
_CHUNK = 16
_NCHTOT = _N // _CHUNK
_NCH_W = -(-_NCHTOT // _NW)
_M = 64
_RC = 5.0

_t = np.linspace(0.0, 1.0, 4001)
_COS_C = [float(c) for c in np.polyfit(_t ** 2, np.cos(np.pi * _t), 6)[::-1]]

_C3B = [
    0.238732414637843, 0.119366207318922, 0.119366207318922,
    0.099471839432435, 0.596831036594608, 0.596831036594608, 0.149207759148652, 0.149207759148652,
    0.139260575205408, 0.104445431404056, 0.104445431404056, 1.044454314040563, 1.044454314040563,
    0.174075719006761, 0.174075719006761,
    0.011190581936149, 0.223811638722978, 0.223811638722978, 0.111905819361489, 0.111905819361489,
    1.566681471060845, 1.566681471060845, 0.195835183882606, 0.195835183882606,
]


_XM = 0x5A5A5A5A


def _sc_body(pos_h, typ_h, nbr_h, ox_h, oy_h, oz_h, ct_h, out_h,
             pos_v, typ_v, ct_v, ctf_v, nbr_v, ox_v, oy_v, oz_v, s_v, ob_v,
             buf_v, sem):
    wid = lax.axis_index("s") * 2 + lax.axis_index("c")
    pltpu.sync_copy(pos_h, pos_v)
    pltpu.sync_copy(typ_h, typ_v)
    pltpu.sync_copy(ct_h, ct_v)
    xm = jnp.full((16,), _XM, jnp.int32)
    for r in range(32):
        ctf_v[pl.ds(r * 16, 16)] = plsc.bitcast(
            ct_v[pl.ds(r * 16, 16)] ^ xm, jnp.float32)
    lanes = lax.broadcasted_iota(jnp.int32, (16,), 0)
    lanes64 = lanes * _M
    lanes192 = lanes * (_M * 3)
    lanes16 = lanes * 16
    zero_f = jnp.zeros((16,), jnp.float32)

    def chunk_body(ci, carry):
        g = wid * _NCH_W + ci

        @pl.when(g < _NCHTOT)
        def _do_chunk():
            _chunk(g)
        return carry

    def _chunk(g):
        base = g * _CHUNK
        h1 = pltpu.async_copy(nbr_h.at[pl.ds(base * _M, _CHUNK * _M)], nbr_v, sem)
        h2 = pltpu.async_copy(ox_h.at[pl.ds(base * _M, _CHUNK * _M)], ox_v, sem)
        h3 = pltpu.async_copy(oy_h.at[pl.ds(base * _M, _CHUNK * _M)], oy_v, sem)
        h4 = pltpu.async_copy(oz_h.at[pl.ds(base * _M, _CHUNK * _M)], oz_v, sem)
        gidx = base + lanes
        t_i = plsc.load_gather(typ_v, [gidx])
        gidx3 = gidx * 3
        pix = plsc.bitcast(plsc.load_gather(pos_v, [gidx3]) ^ xm, jnp.float32)
        piy = plsc.bitcast(plsc.load_gather(pos_v, [gidx3 + 1]) ^ xm, jnp.float32)
        piz = plsc.bitcast(plsc.load_gather(pos_v, [gidx3 + 2]) ^ xm, jnp.float32)
        ti_off = t_i * 128
        for row in range(96):
            s_v[row, :] = zero_f
        h1.wait()
        h2.wait()
        h3.wait()
        h4.wait()

        def edge(j):
            n = plsc.load_gather(nbr_v, [lanes64 + j]) ^ xm
            t_j = plsc.load_gather(typ_v, [n])
            n3 = n * 3
            pjx = plsc.bitcast(plsc.load_gather(pos_v, [n3]) ^ xm, jnp.float32)
            pjy = plsc.bitcast(plsc.load_gather(pos_v, [n3 + 1]) ^ xm, jnp.float32)
            pjz = plsc.bitcast(plsc.load_gather(pos_v, [n3 + 2]) ^ xm, jnp.float32)
            oidx = lanes64 + j
            ox = plsc.bitcast(plsc.load_gather(ox_v, [oidx]) ^ xm, jnp.float32)
            oy = plsc.bitcast(plsc.load_gather(oy_v, [oidx]) ^ xm, jnp.float32)
            oz = plsc.bitcast(plsc.load_gather(oz_v, [oidx]) ^ xm, jnp.float32)
            dx = pjx + ox - pix
            dy = pjy + oy - piy
            dz = pjz + oz - piz
            r2 = jnp.maximum(dx * dx + dy * dy + dz * dz, 1e-24)
            y = plsc.bitcast(jnp.int32(0x5F3759DF) - (plsc.bitcast(r2, jnp.int32) >> 1),
                             jnp.float32)
            h = 0.5 * r2
            y = y * (1.5 - h * y * y)
            y = y * (1.5 - h * y * y)
            y = y * (1.5 - h * y * y)
            t = (r2 * y) * (1.0 / _RC)
            tcl = jnp.minimum(t, 1.0)
            uu = tcl * tcl
            cp = jnp.full((16,), _COS_C[6], jnp.float32)
            for c in _COS_C[5::-1]:
                cp = cp * uu + c
            fh = jnp.where(t < 1.0, 0.25 * cp + 0.25, 0.0)
            xc = 2.0 * (t - 1.0) * (t - 1.0) - 1.0
            gk = [2.0 * fh, (xc + 1.0) * fh]
            fm2 = jnp.ones((16,), jnp.float32)
            fm1 = xc
            for _k in range(2, 8):
                fnk = 2.0 * xc * fm1 - fm2
                gk.append((fnk + 1.0) * fh)
                fm2, fm1 = fm1, fnk
            pb = ti_off + t_j * 32
            gs = []
            for dsc in range(4):
                acc_g = None
                for k in range(8):
                    cv = plsc.load_gather(ctf_v, [pb + (dsc * 8 + k)])
                    acc_g = cv * gk[k] if acc_g is None else acc_g + cv * gk[k]
                gs.append(acc_g)
            ux, uy, uz = dx * y, dy * y, dz * y
            xr2 = ux * ux - uy * uy
            xi2 = 2.0 * ux * uy
            xr3 = xr2 * ux - xi2 * uy
            xi3 = xr2 * uy + xi2 * ux
            xr4 = xr3 * ux - xi3 * uy
            xi4 = xr3 * uy + xi3 * ux
            z2 = uz * uz
            z3 = z2 * uz
            z4 = z2 * z2
            B = [None] * 24
            B[0] = uz
            B[1] = ux
            B[2] = uy
            B[3] = 3.0 * z2 - 1.0
            B[4] = uz * ux
            B[5] = uz * uy
            B[6] = xr2
            B[7] = xi2
            B[8] = 5.0 * z3 - 3.0 * uz
            zf = 5.0 * z2 - 1.0
            B[9] = zf * ux
            B[10] = zf * uy
            B[11] = uz * xr2
            B[12] = uz * xi2
            B[13] = xr3
            B[14] = xi3
            B[15] = 35.0 * z4 - 30.0 * z2 + 3.0
            zf = 7.0 * z3 - 3.0 * uz
            B[16] = zf * ux
            B[17] = zf * uy
            zf = 7.0 * z2 - 1.0
            B[18] = zf * xr2
            B[19] = zf * xi2
            B[20] = uz * xr3
            B[21] = uz * xi3
            B[22] = xr4
            B[23] = xi4
            return gs, B

        @plsc.parallel_loop(0, _M, step=1, unroll=2)
        def s1_body(j):
            gs, B = edge(j)
            for cc in range(24):
                buf_v[cc, j, :] = B[cc]
            for dsc in range(4):
                buf_v[24 + dsc, j, :] = gs[dsc]

        @plsc.parallel_loop(0, _M, step=4, unroll=1)
        def s2_body(j0):
            gv = [[buf_v[24 + dsc, j0 + u, :] for dsc in range(4)]
                  for u in range(4)]
            for cc in range(24):
                b0 = buf_v[cc, j0, :]
                b1 = buf_v[cc, j0 + 1, :]
                b2 = buf_v[cc, j0 + 2, :]
                b3 = buf_v[cc, j0 + 3, :]
                for dsc in range(4):
                    prod = ((gv[0][dsc] * b0 + gv[1][dsc] * b1)
                            + (gv[2][dsc] * b2 + gv[3][dsc] * b3))
                    plsc.addupdate(s_v.at[dsc * 24 + cc], prod)

        for dsc in range(4):
            for L in range(1, 5):
                st = L * L - 1
                qv = None
                for m in range(2 * L + 1):
                    sv = s_v[dsc * 24 + st + m, :]
                    term = _C3B[st + m] * (sv * sv)
                    qv = term if qv is None else qv + term
                col = jnp.full((16,), dsc * 4 + (L - 1), jnp.int32)
                plsc.store_scatter(ob_v, [lanes16 + col],
                                   plsc.bitcast(qv, jnp.int32) ^ xm)
        pltpu.sync_copy(ob_v, out_h.at[pl.ds(base * 16, _CHUNK * 16)])

    lax.fori_loop(0, _NCH_W, chunk_body, 0)


_sc_call = functools.partial(
    pl.kernel,
    out_type=jax.ShapeDtypeStruct((_N * 16,), jnp.int32),
    mesh=plsc.VectorSubcoreMesh(core_axis_name="c", subcore_axis_name="s"),
    compiler_params=pltpu.CompilerParams(needs_layout_passes=False,
                                         use_tc_tiling_on_sc=False),
    scratch_types=[
        pltpu.VMEM((_N * 3,), jnp.int32),
        pltpu.VMEM((_N,), jnp.int32),
        pltpu.VMEM((512,), jnp.int32),
        pltpu.VMEM((512,), jnp.float32),
        pltpu.VMEM((_CHUNK * _M,), jnp.int32),
        pltpu.VMEM((_CHUNK * _M,), jnp.int32),
        pltpu.VMEM((_CHUNK * _M,), jnp.int32),
        pltpu.VMEM((_CHUNK * _M,), jnp.int32),
        pltpu.VMEM((96, 16), jnp.float32),
        pltpu.VMEM((_CHUNK * 16,), jnp.int32),
        pltpu.VMEM((28, _M, 16), jnp.float32),
        pltpu.SemaphoreType.DMA,
    ],
)(_sc_body)


def kernel(positions, types, neighbors, neighbor_offsets, c_table):
    m = jnp.int32(_XM)
    pos_x = lax.bitcast_convert_type(positions.astype(jnp.float32),
                                     jnp.int32).reshape(-1) ^ m
    off_i = lax.bitcast_convert_type(neighbor_offsets.astype(jnp.float32),
                                     jnp.int32)
    ox_x = off_i[:, :, 0].reshape(-1) ^ m
    oy_x = off_i[:, :, 1].reshape(-1) ^ m
    oz_x = off_i[:, :, 2].reshape(-1) ^ m
    nbr_x = neighbors.astype(jnp.int32).reshape(-1) ^ m
    ct_x = lax.bitcast_convert_type(c_table.astype(jnp.float32),
                                    jnp.int32).reshape(-1) ^ m
    out = _sc_call(pos_x, types.astype(jnp.int32), nbr_x, ox_x, oy_x, oz_x, ct_x)
    return lax.bitcast_convert_type(out ^ m, jnp.float32).reshape(_N, 16)


# speedup vs baseline: 1.5141x; 1.5141x over previous
import functools

import numpy as np
import jax
import jax.numpy as jnp
from jax import lax
from jax.experimental import pallas as pl
from jax.experimental.pallas import tpu as pltpu
from jax.experimental.pallas import tpu_sc as plsc

_N = 10000
_NW = 32
_CHUNK = 16
_NCHTOT = _N // _CHUNK
_NCH_W = -(-_NCHTOT // _NW)
_M = 64
_RC = 5.0

_t = np.linspace(0.0, 1.0, 4001)
_COS_C = [float(c) for c in np.polyfit(_t ** 2, np.cos(np.pi * _t), 6)[::-1]]

_C3B = [
    0.238732414637843, 0.119366207318922, 0.119366207318922,
    0.099471839432435, 0.596831036594608, 0.596831036594608, 0.149207759148652, 0.149207759148652,
    0.139260575205408, 0.104445431404056, 0.104445431404056, 1.044454314040563, 1.044454314040563,
    0.174075719006761, 0.174075719006761,
    0.011190581936149, 0.223811638722978, 0.223811638722978, 0.111905819361489, 0.111905819361489,
    1.566681471060845, 1.566681471060845, 0.195835183882606, 0.195835183882606,
]


_XM = 0x5A5A5A5A


def _sc_body(pos_h, typ_h, nbr_h, ox_h, oy_h, oz_h, ct_h, out_h,
             pos_v, typ_v, ct_v, ctf_v, nbr_v, ox_v, oy_v, oz_v, s_v, ob_v,
             buf_v, sem):
    wid = lax.axis_index("s") * 2 + lax.axis_index("c")
    pltpu.sync_copy(pos_h, pos_v)
    pltpu.sync_copy(typ_h, typ_v)
    pltpu.sync_copy(ct_h, ct_v)
    xm = jnp.full((16,), _XM, jnp.int32)
    for r in range(33):
        ctf_v[pl.ds(r * 16, 16)] = plsc.bitcast(
            ct_v[pl.ds(r * 16, 16)] ^ xm, jnp.float32)
    lanes = lax.broadcasted_iota(jnp.int32, (16,), 0)
    lanes64 = lanes * _M
    lanes192 = lanes * (_M * 3)
    lanes16 = lanes * 16
    zero_f = jnp.zeros((16,), jnp.float32)

    def chunk_body(ci, carry):
        g = wid * _NCH_W + ci

        @pl.when(g < _NCHTOT)
        def _do_chunk():
            _chunk(g)
        return carry

    def _chunk(g):
        base = g * _CHUNK
        h1 = pltpu.async_copy(nbr_h.at[pl.ds(base * _M, _CHUNK * _M)], nbr_v, sem)
        h2 = pltpu.async_copy(ox_h.at[pl.ds(base * _M, _CHUNK * _M)], ox_v, sem)
        h3 = pltpu.async_copy(oy_h.at[pl.ds(base * _M, _CHUNK * _M)], oy_v, sem)
        h4 = pltpu.async_copy(oz_h.at[pl.ds(base * _M, _CHUNK * _M)], oz_v, sem)
        gidx = base + lanes
        t_i = plsc.load_gather(typ_v, [gidx])
        gidx3 = gidx * 3
        pix = plsc.bitcast(plsc.load_gather(pos_v, [gidx3]) ^ xm, jnp.float32)
        piy = plsc.bitcast(plsc.load_gather(pos_v, [gidx3 + 1]) ^ xm, jnp.float32)
        piz = plsc.bitcast(plsc.load_gather(pos_v, [gidx3 + 2]) ^ xm, jnp.float32)
        ti_off = t_i * 132
        for row in range(96):
            s_v[row, :] = zero_f
        h1.wait()
        h2.wait()
        h3.wait()
        h4.wait()

        def edge(j):
            jm = j * 16 + lanes
            n = plsc.load_gather(nbr_v, [jm]) ^ xm
            t_j = plsc.load_gather(typ_v, [n])
            n3 = n * 3
            pjx = plsc.bitcast(plsc.load_gather(pos_v, [n3]) ^ xm, jnp.float32)
            pjy = plsc.bitcast(plsc.load_gather(pos_v, [n3 + 1]) ^ xm, jnp.float32)
            pjz = plsc.bitcast(plsc.load_gather(pos_v, [n3 + 2]) ^ xm, jnp.float32)
            ox = plsc.bitcast(plsc.load_gather(ox_v, [jm]) ^ xm, jnp.float32)
            oy = plsc.bitcast(plsc.load_gather(oy_v, [jm]) ^ xm, jnp.float32)
            oz = plsc.bitcast(plsc.load_gather(oz_v, [jm]) ^ xm, jnp.float32)
            dx = pjx + ox - pix
            dy = pjy + oy - piy
            dz = pjz + oz - piz
            r2 = jnp.maximum(dx * dx + dy * dy + dz * dz, 1e-24)
            y = plsc.bitcast(jnp.int32(0x5F3759DF) - (plsc.bitcast(r2, jnp.int32) >> 1),
                             jnp.float32)
            h = 0.5 * r2
            y = y * (1.5 - h * y * y)
            y = y * (1.5 - h * y * y)
            y = y * (1.5 - h * y * y)
            t = (r2 * y) * (1.0 / _RC)
            tcl = jnp.minimum(t, 1.0)
            uu = tcl * tcl
            cp = jnp.full((16,), _COS_C[6], jnp.float32)
            for c in _COS_C[5::-1]:
                cp = cp * uu + c
            fh = jnp.where(t < 1.0, 0.25 * cp + 0.25, 0.0)
            xc = 2.0 * (t - 1.0) * (t - 1.0) - 1.0
            gk = [2.0 * fh, (xc + 1.0) * fh]
            fm2 = jnp.ones((16,), jnp.float32)
            fm1 = xc
            for _k in range(2, 8):
                fnk = 2.0 * xc * fm1 - fm2
                gk.append((fnk + 1.0) * fh)
                fm2, fm1 = fm1, fnk
            pb = ti_off + t_j * 33
            gs = []
            for dsc in range(4):
                acc_g = None
                for k in range(8):
                    cv = plsc.load_gather(ctf_v, [pb + (dsc * 8 + k)])
                    acc_g = cv * gk[k] if acc_g is None else acc_g + cv * gk[k]
                gs.append(acc_g)
            ux, uy, uz = dx * y, dy * y, dz * y
            xr2 = ux * ux - uy * uy
            xi2 = 2.0 * ux * uy
            xr3 = xr2 * ux - xi2 * uy
            xi3 = xr2 * uy + xi2 * ux
            xr4 = xr3 * ux - xi3 * uy
            xi4 = xr3 * uy + xi3 * ux
            z2 = uz * uz
            z3 = z2 * uz
            z4 = z2 * z2
            B = [None] * 24
            B[0] = uz
            B[1] = ux
            B[2] = uy
            B[3] = 3.0 * z2 - 1.0
            B[4] = uz * ux
            B[5] = uz * uy
            B[6] = xr2
            B[7] = xi2
            B[8] = 5.0 * z3 - 3.0 * uz
            zf = 5.0 * z2 - 1.0
            B[9] = zf * ux
            B[10] = zf * uy
            B[11] = uz * xr2
            B[12] = uz * xi2
            B[13] = xr3
            B[14] = xi3
            B[15] = 35.0 * z4 - 30.0 * z2 + 3.0
            zf = 7.0 * z3 - 3.0 * uz
            B[16] = zf * ux
            B[17] = zf * uy
            zf = 7.0 * z2 - 1.0
            B[18] = zf * xr2
            B[19] = zf * xi2
            B[20] = uz * xr3
            B[21] = uz * xi3
            B[22] = xr4
            B[23] = xi4
            return gs, B

        @plsc.parallel_loop(0, _M, step=1, unroll=2)
        def s1_body(j):
            gs, B = edge(j)
            for cc in range(24):
                buf_v[cc, j, :] = B[cc]
            for dsc in range(4):
                buf_v[24 + dsc, j, :] = gs[dsc]

        @plsc.parallel_loop(0, _M, step=4, unroll=1)
        def s2_body(j0):
            gv = [[buf_v[24 + dsc, j0 + u, :] for dsc in range(4)]
                  for u in range(4)]
            for cc in range(24):
                b0 = buf_v[cc, j0, :]
                b1 = buf_v[cc, j0 + 1, :]
                b2 = buf_v[cc, j0 + 2, :]
                b3 = buf_v[cc, j0 + 3, :]
                for dsc in range(4):
                    prod = ((gv[0][dsc] * b0 + gv[1][dsc] * b1)
                            + (gv[2][dsc] * b2 + gv[3][dsc] * b3))
                    plsc.addupdate(s_v.at[dsc * 24 + cc], prod)

        for dsc in range(4):
            for L in range(1, 5):
                st = L * L - 1
                qv = None
                for m in range(2 * L + 1):
                    sv = s_v[dsc * 24 + st + m, :]
                    term = _C3B[st + m] * (sv * sv)
                    qv = term if qv is None else qv + term
                col = jnp.full((16,), dsc * 4 + (L - 1), jnp.int32)
                plsc.store_scatter(ob_v, [lanes16 + col],
                                   plsc.bitcast(qv, jnp.int32) ^ xm)
        pltpu.sync_copy(ob_v, out_h.at[pl.ds(base * 16, _CHUNK * 16)])

    lax.fori_loop(0, _NCH_W, chunk_body, 0)


_sc_call = functools.partial(
    pl.kernel,
    out_type=jax.ShapeDtypeStruct((_N * 16,), jnp.int32),
    mesh=plsc.VectorSubcoreMesh(core_axis_name="c", subcore_axis_name="s"),
    compiler_params=pltpu.CompilerParams(needs_layout_passes=False,
                                         use_tc_tiling_on_sc=False),
    scratch_types=[
        pltpu.VMEM((_N * 3,), jnp.int32),
        pltpu.VMEM((_N,), jnp.int32),
        pltpu.VMEM((528,), jnp.int32),
        pltpu.VMEM((528,), jnp.float32),
        pltpu.VMEM((_CHUNK * _M,), jnp.int32),
        pltpu.VMEM((_CHUNK * _M,), jnp.int32),
        pltpu.VMEM((_CHUNK * _M,), jnp.int32),
        pltpu.VMEM((_CHUNK * _M,), jnp.int32),
        pltpu.VMEM((96, 16), jnp.float32),
        pltpu.VMEM((_CHUNK * 16,), jnp.int32),
        pltpu.VMEM((28, _M, 16), jnp.float32),
        pltpu.SemaphoreType.DMA,
    ],
)(_sc_body)


def kernel(positions, types, neighbors, neighbor_offsets, c_table):
    m = jnp.int32(_XM)
    pos_x = lax.bitcast_convert_type(positions.astype(jnp.float32),
                                     jnp.int32).reshape(-1) ^ m
    off_i = lax.bitcast_convert_type(neighbor_offsets.astype(jnp.float32),
                                     jnp.int32)

    def _cm(plane):
        return (plane.reshape(_NCHTOT, _CHUNK, _M)
                .transpose(0, 2, 1).reshape(-1) ^ m)

    ox_x = _cm(off_i[:, :, 0])
    oy_x = _cm(off_i[:, :, 1])
    oz_x = _cm(off_i[:, :, 2])
    nbr_x = _cm(neighbors.astype(jnp.int32))
    ct_i = lax.bitcast_convert_type(c_table.astype(jnp.float32),
                                    jnp.int32).reshape(16, 32)
    ct_x = jnp.pad(ct_i, ((0, 0), (0, 1))).reshape(-1) ^ m
    out = _sc_call(pos_x, types.astype(jnp.int32), nbr_x, ox_x, oy_x, oz_x, ct_x)
    return lax.bitcast_convert_type(out ^ m, jnp.float32).reshape(_N, 16)


# speedup vs baseline: 1.7786x; 1.1747x over previous
import functools

import numpy as np
import jax
import jax.numpy as jnp
from jax import lax
from jax.experimental import pallas as pl
from jax.experimental.pallas import tpu as pltpu
from jax.experimental.pallas import tpu_sc as plsc

_N = 10000
_NW = 32
_CHUNK = 16
_NCHTOT = _N // _CHUNK
_NCH_W = -(-_NCHTOT // _NW)
_M = 64
_RC = 5.0

_t = np.linspace(0.0, 1.0, 4001)
_COS_C = [float(c) for c in np.polyfit(_t ** 2, np.cos(np.pi * _t), 6)[::-1]]

_C3B = [
    0.238732414637843, 0.119366207318922, 0.119366207318922,
    0.099471839432435, 0.596831036594608, 0.596831036594608, 0.149207759148652, 0.149207759148652,
    0.139260575205408, 0.104445431404056, 0.104445431404056, 1.044454314040563, 1.044454314040563,
    0.174075719006761, 0.174075719006761,
    0.011190581936149, 0.223811638722978, 0.223811638722978, 0.111905819361489, 0.111905819361489,
    1.566681471060845, 1.566681471060845, 0.195835183882606, 0.195835183882606,
]


_XM = 0x5A5A5A5A


def _sc_body(pos_h, typ_h, big_h, ct_h, out_h,
             pos_v, typ_v, ct_v, ctf_v, big_v, s_v, ob_v,
             buf_v, sem):
    wid = lax.axis_index("s") * 2 + lax.axis_index("c")
    pltpu.sync_copy(pos_h, pos_v)
    pltpu.sync_copy(typ_h, typ_v)
    pltpu.sync_copy(ct_h, ct_v)
    xm = jnp.full((16,), _XM, jnp.int32)
    for r in range(33):
        ctf_v[pl.ds(r * 16, 16)] = plsc.bitcast(
            ct_v[pl.ds(r * 16, 16)] ^ xm, jnp.float32)
    lanes = lax.broadcasted_iota(jnp.int32, (16,), 0)
    lanes64 = lanes * _M
    lanes192 = lanes * (_M * 3)
    lanes16 = lanes * 16
    zero_f = jnp.zeros((16,), jnp.float32)

    def chunk_body(ci, carry):
        g = wid * _NCH_W + ci

        @pl.when(g < _NCHTOT)
        def _do_chunk():
            _chunk(g)
        return carry

    def _chunk(g):
        base = g * _CHUNK
        h1 = pltpu.async_copy(big_h.at[pl.ds(g * (4 * _CHUNK * _M),
                                             4 * _CHUNK * _M)], big_v, sem)
        gidx = base + lanes
        t_i = plsc.load_gather(typ_v, [gidx])
        gidx3 = gidx * 3
        pix = plsc.bitcast(plsc.load_gather(pos_v, [gidx3]) ^ xm, jnp.float32)
        piy = plsc.bitcast(plsc.load_gather(pos_v, [gidx3 + 1]) ^ xm, jnp.float32)
        piz = plsc.bitcast(plsc.load_gather(pos_v, [gidx3 + 2]) ^ xm, jnp.float32)
        ti_off = t_i * 132
        for row in range(96):
            s_v[row, :] = zero_f
        h1.wait()

        def edge(j):
            jm = j * 16 + lanes
            n = plsc.load_gather(big_v, [jm]) ^ xm
            t_j = plsc.load_gather(typ_v, [n])
            n3 = n * 3
            pjx = plsc.bitcast(plsc.load_gather(pos_v, [n3]) ^ xm, jnp.float32)
            pjy = plsc.bitcast(plsc.load_gather(pos_v, [n3 + 1]) ^ xm, jnp.float32)
            pjz = plsc.bitcast(plsc.load_gather(pos_v, [n3 + 2]) ^ xm, jnp.float32)
            ox = plsc.bitcast(plsc.load_gather(big_v, [jm + 1024]) ^ xm, jnp.float32)
            oy = plsc.bitcast(plsc.load_gather(big_v, [jm + 2048]) ^ xm, jnp.float32)
            oz = plsc.bitcast(plsc.load_gather(big_v, [jm + 3072]) ^ xm, jnp.float32)
            dx = pjx + ox - pix
            dy = pjy + oy - piy
            dz = pjz + oz - piz
            r2 = jnp.maximum(dx * dx + dy * dy + dz * dz, 1e-24)
            y = plsc.bitcast(jnp.int32(0x5F3759DF) - (plsc.bitcast(r2, jnp.int32) >> 1),
                             jnp.float32)
            h = 0.5 * r2
            y = y * (1.5 - h * y * y)
            y = y * (1.5 - h * y * y)
            y = y * (1.5 - h * y * y)
            t = (r2 * y) * (1.0 / _RC)
            tcl = jnp.minimum(t, 1.0)
            uu = tcl * tcl
            cp = jnp.full((16,), _COS_C[6], jnp.float32)
            for c in _COS_C[5::-1]:
                cp = cp * uu + c
            fh = jnp.where(t < 1.0, 0.25 * cp + 0.25, 0.0)
            xc = 2.0 * (t - 1.0) * (t - 1.0) - 1.0
            gk = [2.0 * fh, (xc + 1.0) * fh]
            fm2 = jnp.ones((16,), jnp.float32)
            fm1 = xc
            for _k in range(2, 8):
                fnk = 2.0 * xc * fm1 - fm2
                gk.append((fnk + 1.0) * fh)
                fm2, fm1 = fm1, fnk
            pb = ti_off + t_j * 33
            gs = []
            for dsc in range(4):
                acc_g = None
                for k in range(8):
                    cv = plsc.load_gather(ctf_v, [pb + (dsc * 8 + k)])
                    acc_g = cv * gk[k] if acc_g is None else acc_g + cv * gk[k]
                gs.append(acc_g)
            ux, uy, uz = dx * y, dy * y, dz * y
            xr2 = ux * ux - uy * uy
            xi2 = 2.0 * ux * uy
            xr3 = xr2 * ux - xi2 * uy
            xi3 = xr2 * uy + xi2 * ux
            xr4 = xr3 * ux - xi3 * uy
            xi4 = xr3 * uy + xi3 * ux
            z2 = uz * uz
            z3 = z2 * uz
            z4 = z2 * z2
            B = [None] * 24
            B[0] = uz
            B[1] = ux
            B[2] = uy
            B[3] = 3.0 * z2 - 1.0
            B[4] = uz * ux
            B[5] = uz * uy
            B[6] = xr2
            B[7] = xi2
            B[8] = 5.0 * z3 - 3.0 * uz
            zf = 5.0 * z2 - 1.0
            B[9] = zf * ux
            B[10] = zf * uy
            B[11] = uz * xr2
            B[12] = uz * xi2
            B[13] = xr3
            B[14] = xi3
            B[15] = 35.0 * z4 - 30.0 * z2 + 3.0
            zf = 7.0 * z3 - 3.0 * uz
            B[16] = zf * ux
            B[17] = zf * uy
            zf = 7.0 * z2 - 1.0
            B[18] = zf * xr2
            B[19] = zf * xi2
            B[20] = uz * xr3
            B[21] = uz * xi3
            B[22] = xr4
            B[23] = xi4
            return gs, B

        @plsc.parallel_loop(0, _M, step=1, unroll=2)
        def s1_body(j):
            gs, B = edge(j)
            for cc in range(24):
                buf_v[cc, j, :] = B[cc]
            for dsc in range(4):
                buf_v[24 + dsc, j, :] = gs[dsc]

        @plsc.parallel_loop(0, _M, step=4, unroll=1)
        def s2_body(j0):
            gv = [[buf_v[24 + dsc, j0 + u, :] for dsc in range(4)]
                  for u in range(4)]
            for cc in range(24):
                b0 = buf_v[cc, j0, :]
                b1 = buf_v[cc, j0 + 1, :]
                b2 = buf_v[cc, j0 + 2, :]
                b3 = buf_v[cc, j0 + 3, :]
                for dsc in range(4):
                    prod = ((gv[0][dsc] * b0 + gv[1][dsc] * b1)
                            + (gv[2][dsc] * b2 + gv[3][dsc] * b3))
                    plsc.addupdate(s_v.at[dsc * 24 + cc], prod)

        for dsc in range(4):
            for L in range(1, 5):
                st = L * L - 1
                qv = None
                for m in range(2 * L + 1):
                    sv = s_v[dsc * 24 + st + m, :]
                    term = _C3B[st + m] * (sv * sv)
                    qv = term if qv is None else qv + term
                col = jnp.full((16,), dsc * 4 + (L - 1), jnp.int32)
                plsc.store_scatter(ob_v, [lanes16 + col],
                                   plsc.bitcast(qv, jnp.int32) ^ xm)
        pltpu.sync_copy(ob_v, out_h.at[pl.ds(base * 16, _CHUNK * 16)])

    lax.fori_loop(0, _NCH_W, chunk_body, 0)


_sc_call = functools.partial(
    pl.kernel,
    out_type=jax.ShapeDtypeStruct((_N * 16,), jnp.int32),
    mesh=plsc.VectorSubcoreMesh(core_axis_name="c", subcore_axis_name="s"),
    compiler_params=pltpu.CompilerParams(needs_layout_passes=False,
                                         use_tc_tiling_on_sc=False),
    scratch_types=[
        pltpu.VMEM((_N * 3,), jnp.int32),
        pltpu.VMEM((_N,), jnp.int32),
        pltpu.VMEM((528,), jnp.int32),
        pltpu.VMEM((528,), jnp.float32),
        pltpu.VMEM((4 * _CHUNK * _M,), jnp.int32),
        pltpu.VMEM((96, 16), jnp.float32),
        pltpu.VMEM((_CHUNK * 16,), jnp.int32),
        pltpu.VMEM((28, _M, 16), jnp.float32),
        pltpu.SemaphoreType.DMA,
    ],
)(_sc_body)


def kernel(positions, types, neighbors, neighbor_offsets, c_table):
    m = jnp.int32(_XM)
    pos_x = lax.bitcast_convert_type(positions.astype(jnp.float32),
                                     jnp.int32).reshape(-1) ^ m
    off_i = lax.bitcast_convert_type(neighbor_offsets.astype(jnp.float32),
                                     jnp.int32)

    def _cm(plane):
        return (plane.reshape(_NCHTOT, _CHUNK, _M)
                .transpose(0, 2, 1).reshape(-1) ^ m)

    big_x = jnp.stack([_cm(neighbors.astype(jnp.int32)).reshape(_NCHTOT, -1),
                       _cm(off_i[:, :, 0]).reshape(_NCHTOT, -1),
                       _cm(off_i[:, :, 1]).reshape(_NCHTOT, -1),
                       _cm(off_i[:, :, 2]).reshape(_NCHTOT, -1)],
                      axis=1).reshape(-1)
    ct_i = lax.bitcast_convert_type(c_table.astype(jnp.float32),
                                    jnp.int32).reshape(16, 32)
    ct_x = jnp.pad(ct_i, ((0, 0), (0, 1))).reshape(-1) ^ m
    out = _sc_call(pos_x, types.astype(jnp.int32), big_x, ct_x)
    return lax.bitcast_convert_type(out ^ m, jnp.float32).reshape(_N, 16)
